# baseline (speedup 1.0000x reference)
import jax
import jax.numpy as jnp
from jax import lax
from jax.experimental import pallas as pl
from jax.experimental.pallas import tpu as pltpu
from jax.experimental.pallas import tpu_sc as plsc

NUM_CLASS = 100000
N_CLS_CTX = 4
CTX_DIM = 512
N_PRE = 5
N_SUF = 68
SEQ = 77
BATCH = 4096

_NC = 2
_NS = 16
_NW = _NC * _NS
_BPW = BATCH // _NW
_PHASES = 8
_RPP = _BPW // _PHASES


def _sc_body(label_hbm, table_hbm, prefix_hbm, suffix_hbm, out_hbm,
             lidx_v, pre_v, suf_v, gbuf, gsem, wsem, hsem):
    wid = lax.axis_index("s") * _NC + lax.axis_index("c")
    base = wid * _BPW

    pltpu.sync_copy(label_hbm.at[pl.ds(base, _BPW)], lidx_v)
    pltpu.sync_copy(prefix_hbm.at[0], pre_v)
    pltpu.sync_copy(suffix_hbm.at[0], suf_v)

    def pswrite(i, carry):
        b = base + i
        pltpu.async_copy(pre_v, out_hbm.at[b, pl.ds(0, N_PRE)], wsem)
        pltpu.async_copy(suf_v, out_hbm.at[b, pl.ds(N_PRE + N_CLS_CTX, N_SUF)],
                         wsem)
        return carry

    g_h = {}
    for p in range(_PHASES):
        pb = p % 2
        if p >= 2:
            def hdrain(i, carry, pb=pb):
                pltpu.make_async_copy(gbuf.at[pb, 0],
                                      out_hbm.at[0, pl.ds(N_PRE, N_CLS_CTX)],
                                      hsem).wait()
                return carry
            lax.fori_loop(0, _RPP, hdrain, 0)
        g_h[p] = pltpu.async_copy(table_hbm.at[lidx_v.at[pl.ds(p * _RPP, _RPP)]],
                                  gbuf.at[pb], gsem)
        lax.fori_loop(p * _RPP, (p + 1) * _RPP, pswrite, 0)
        g_h.pop(p).wait()

        def hwrite(i, carry, p=p, pb=pb):
            b = base + p * _RPP + i
            pltpu.async_copy(gbuf.at[pb, i], out_hbm.at[b, pl.ds(N_PRE, N_CLS_CTX)],
                             hsem)
            return carry

        lax.fori_loop(0, _RPP, hwrite, 0)

    def hdrain2(i, carry):
        pltpu.make_async_copy(gbuf.at[0, 0],
                              out_hbm.at[0, pl.ds(N_PRE, N_CLS_CTX)],
                              hsem).wait()
        return carry

    lax.fori_loop(0, 2 * _RPP, hdrain2, 0)

    def psdrain(i, carry):
        pltpu.make_async_copy(pre_v, out_hbm.at[0, pl.ds(0, N_PRE)], wsem).wait()
        pltpu.make_async_copy(suf_v, out_hbm.at[0, pl.ds(N_PRE + N_CLS_CTX, N_SUF)],
                              wsem).wait()
        return carry

    lax.fori_loop(0, _BPW, psdrain, 0)


def kernel(label, cls_ctx, token_prefix, token_suffix):
    mesh = plsc.VectorSubcoreMesh(core_axis_name="c", subcore_axis_name="s")
    f = pl.kernel(
        _sc_body,
        out_type=jax.ShapeDtypeStruct((BATCH, SEQ, CTX_DIM), jnp.float32),
        mesh=mesh,
        compiler_params=pltpu.CompilerParams(use_tc_tiling_on_sc=False),
        scratch_types=[
            pltpu.VMEM((_BPW,), jnp.int32),
            pltpu.VMEM((N_PRE, CTX_DIM), jnp.float32),
            pltpu.VMEM((N_SUF, CTX_DIM), jnp.float32),
            pltpu.VMEM((2, _RPP, N_CLS_CTX, CTX_DIM), jnp.float32),
            pltpu.SemaphoreType.DMA,
            pltpu.SemaphoreType.DMA,
            pltpu.SemaphoreType.DMA,
        ],
    )
    return f(label, cls_ctx, token_prefix, token_suffix)


# speedup vs baseline: 2.1521x; 2.1521x over previous
import jax
import jax.numpy as jnp
from jax import lax
from jax.experimental import pallas as pl
from jax.experimental.pallas import tpu as pltpu
from jax.experimental.pallas import tpu_sc as plsc

NUM_CLASS = 100000
N_CLS_CTX = 4
CTX_DIM = 512
N_PRE = 5
N_SUF = 68
SEQ = 77
BATCH = 4096

_NC = 2
_NS = 16
_NW = _NC * _NS
_BPW = BATCH // _NW
_PHASES = 8
_RPP = _BPW // _PHASES

_BB = 8


def _sc_gather_body(label_hbm, table_hbm, out_hbm, lidx_v, gbuf, gsem, wsem):
    wid = lax.axis_index("s") * _NC + lax.axis_index("c")
    base = wid * _BPW

    pltpu.sync_copy(label_hbm.at[pl.ds(base, _BPW)], lidx_v)

    g_h = {}
    w_n = 0
    for p in range(_PHASES):
        pb = p % 2
        if p >= 2:
            pltpu.make_async_copy(gbuf.at[pb],
                                  out_hbm.at[pl.ds(0, _RPP)], wsem).wait()
            w_n -= 1
        g_h[p] = pltpu.async_copy(
            table_hbm.at[lidx_v.at[pl.ds(p * _RPP, _RPP)]], gbuf.at[pb], gsem)
        g_h.pop(p).wait()
        pltpu.async_copy(gbuf.at[pb],
                         out_hbm.at[pl.ds(base + p * _RPP, _RPP)], wsem)
        w_n += 1
    for _ in range(w_n):
        pltpu.make_async_copy(gbuf.at[0], out_hbm.at[pl.ds(0, _RPP)],
                              wsem).wait()


def _tc_concat_body(g_ref, pre_ref, suf_ref, out_ref):
    pre = pre_ref[0]
    suf = suf_ref[0]
    out_ref[:, 0:N_PRE, :] = jnp.broadcast_to(
        pre[None], (_BB, N_PRE, CTX_DIM))
    out_ref[:, N_PRE:N_PRE + N_CLS_CTX, :] = g_ref[...]
    out_ref[:, N_PRE + N_CLS_CTX:, :] = jnp.broadcast_to(
        suf[None], (_BB, N_SUF, CTX_DIM))


def kernel(label, cls_ctx, token_prefix, token_suffix):
    mesh = plsc.VectorSubcoreMesh(core_axis_name="c", subcore_axis_name="s")
    gathered = pl.kernel(
        _sc_gather_body,
        out_type=jax.ShapeDtypeStruct((BATCH, N_CLS_CTX, CTX_DIM), jnp.float32),
        mesh=mesh,
        scratch_types=[
            pltpu.VMEM((_BPW,), jnp.int32),
            pltpu.VMEM((2, _RPP, N_CLS_CTX, CTX_DIM), jnp.float32),
            pltpu.SemaphoreType.DMA,
            pltpu.SemaphoreType.DMA,
        ],
    )(label, cls_ctx)

    out = pl.pallas_call(
        _tc_concat_body,
        grid=(BATCH // _BB,),
        in_specs=[
            pl.BlockSpec((_BB, N_CLS_CTX, CTX_DIM), lambda i: (i, 0, 0)),
            pl.BlockSpec((1, N_PRE, CTX_DIM), lambda i: (0, 0, 0)),
            pl.BlockSpec((1, N_SUF, CTX_DIM), lambda i: (0, 0, 0)),
        ],
        out_specs=pl.BlockSpec((_BB, SEQ, CTX_DIM), lambda i: (i, 0, 0)),
        out_shape=jax.ShapeDtypeStruct((BATCH, SEQ, CTX_DIM), jnp.float32),
        compiler_params=pltpu.CompilerParams(
            dimension_semantics=("arbitrary",)),
    )(gathered, token_prefix, token_suffix)
    return out


# speedup vs baseline: 7.4570x; 3.4650x over previous
import jax
import jax.numpy as jnp
from jax import lax
from jax.experimental import pallas as pl
from jax.experimental.pallas import tpu as pltpu
from jax.experimental.pallas import tpu_sc as plsc

NUM_CLASS = 100000
N_CLS_CTX = 4
CTX_DIM = 512
N_PRE = 5
N_SUF = 68
SEQ = 77
BATCH = 4096

_NC = 2
_NS = 16
_NW = _NC * _NS
_BPW = BATCH // _NW
_PHASES = 8
_RPP = _BPW // _PHASES

_BB = 128


def _sc_gather_body(label_hbm, table_hbm, out_hbm, lidx_v, gbuf, gsem, wsem):
    wid = lax.axis_index("s") * _NC + lax.axis_index("c")
    base = wid * _BPW

    pltpu.sync_copy(label_hbm.at[pl.ds(base, _BPW)], lidx_v)

    g_h = {}
    w_n = 0
    for p in range(_PHASES):
        pb = p % 2
        if p >= 2:
            pltpu.make_async_copy(gbuf.at[pb],
                                  out_hbm.at[pl.ds(0, _RPP)], wsem).wait()
            w_n -= 1
        g_h[p] = pltpu.async_copy(
            table_hbm.at[lidx_v.at[pl.ds(p * _RPP, _RPP)]], gbuf.at[pb], gsem)
        g_h.pop(p).wait()
        pltpu.async_copy(gbuf.at[pb],
                         out_hbm.at[pl.ds(base + p * _RPP, _RPP)], wsem)
        w_n += 1
    for _ in range(w_n):
        pltpu.make_async_copy(gbuf.at[0], out_hbm.at[pl.ds(0, _RPP)],
                              wsem).wait()


def _tc_concat_body(g_ref, pre_ref, suf_ref, out_ref):
    pre = pre_ref[0]
    suf = suf_ref[0]
    out_ref[0:N_PRE] = jnp.broadcast_to(pre[:, None], (N_PRE, _BB, CTX_DIM))
    for j in range(N_CLS_CTX):
        out_ref[N_PRE + j] = g_ref[:, j, :]
    out_ref[N_PRE + N_CLS_CTX:] = jnp.broadcast_to(
        suf[:, None], (N_SUF, _BB, CTX_DIM))


def kernel(label, cls_ctx, token_prefix, token_suffix):
    mesh = plsc.VectorSubcoreMesh(core_axis_name="c", subcore_axis_name="s")
    gathered = pl.kernel(
        _sc_gather_body,
        out_type=jax.ShapeDtypeStruct((BATCH, N_CLS_CTX, CTX_DIM), jnp.float32),
        mesh=mesh,
        scratch_types=[
            pltpu.VMEM((_BPW,), jnp.int32),
            pltpu.VMEM((2, _RPP, N_CLS_CTX, CTX_DIM), jnp.float32),
            pltpu.SemaphoreType.DMA,
            pltpu.SemaphoreType.DMA,
        ],
    )(label, cls_ctx)

    out_t = pl.pallas_call(
        _tc_concat_body,
        grid=(BATCH // _BB,),
        in_specs=[
            pl.BlockSpec((_BB, N_CLS_CTX, CTX_DIM), lambda i: (i, 0, 0)),
            pl.BlockSpec((1, N_PRE, CTX_DIM), lambda i: (0, 0, 0)),
            pl.BlockSpec((1, N_SUF, CTX_DIM), lambda i: (0, 0, 0)),
        ],
        out_specs=pl.BlockSpec((SEQ, _BB, CTX_DIM), lambda i: (0, i, 0)),
        out_shape=jax.ShapeDtypeStruct((SEQ, BATCH, CTX_DIM), jnp.float32),
        compiler_params=pltpu.CompilerParams(
            dimension_semantics=("arbitrary",)),
    )(gathered, token_prefix, token_suffix)
    return jnp.transpose(out_t, (1, 0, 2))
